# Initial kernel scaffold; baseline (speedup 1.0000x reference)
import jax
import jax.numpy as jnp
from jax.experimental import pallas as pl


def kernel(x, emb_table, W, b):
    raise NotImplementedError("write your pallas kernel here")


# speedup vs baseline: 2.4321x; 2.4321x over previous
import functools

import jax
import jax.numpy as jnp
from jax import lax
from jax.experimental import pallas as pl
from jax.experimental.pallas import tpu as pltpu
from jax.experimental.pallas import tpu_sc as plsc

VOCAB = 1000000
EMBED_DIM = 64
NUM_CLASSES = 1000
BATCH = 16384
SEQLEN = 50

_PAD_CLASSES = 1024
_NC = 2
_NS = 16
_NW = _NC * _NS
_ROWS_PER_W = BATCH // _NW
_PAIRS_PER_W = _ROWS_PER_W // 2
_IDX_PER_GATHER = 2 * SEQLEN
_QV = EMBED_DIM // 16


def _sc_pool_body(x_hbm, tab_hbm, out_hbm, idx_v, buf0, buf1, out_v, sem0, sem1):
    wid = lax.axis_index("s") * _NC + lax.axis_index("c")
    pair_base = wid * _PAIRS_PER_W

    pltpu.sync_copy(x_hbm.at[pl.ds(pair_base, _PAIRS_PER_W), :], idx_v)

    pltpu.async_copy(tab_hbm.at[idx_v.at[0]], buf0, sem0)

    bufs = (buf0, buf1)
    sems = (sem0, sem1)

    def accumulate(buf, j):
        def rbody(r, acc):
            new = []
            for half in range(2):
                for q in range(_QV):
                    v = buf[half * SEQLEN + r, pl.ds(q * 16, 16)]
                    new.append(acc[half * _QV + q] + v)
            return tuple(new)

        zeros = tuple(jnp.zeros((16,), jnp.float32) for _ in range(2 * _QV))
        acc = lax.fori_loop(0, SEQLEN, rbody, zeros)
        scale = jnp.float32(1.0 / SEQLEN)
        for half in range(2):
            for q in range(_QV):
                out_v[2 * j + half, pl.ds(q * 16, 16)] = acc[half * _QV + q] * scale

    @pl.loop(0, _PAIRS_PER_W, step=2)
    def _(g):
        for b in range(2):
            j = g + b
            nxt = 1 - b

            @pl.when(j + 1 < _PAIRS_PER_W)
            def _():
                pltpu.async_copy(tab_hbm.at[idx_v.at[j + 1]], bufs[nxt], sems[nxt])

            pltpu.make_async_copy(tab_hbm.at[idx_v.at[j]], bufs[b], sems[b]).wait()
            accumulate(bufs[b], j)

    pltpu.sync_copy(out_v, out_hbm.at[pl.ds(wid * _ROWS_PER_W, _ROWS_PER_W), :])


def _make_sc_pool():
    mesh = plsc.VectorSubcoreMesh(core_axis_name="c", subcore_axis_name="s")
    return pl.kernel(
        _sc_pool_body,
        out_type=jax.ShapeDtypeStruct((BATCH, EMBED_DIM), jnp.float32),
        mesh=mesh,
        compiler_params=pltpu.CompilerParams(use_tc_tiling_on_sc=False),
        scratch_types=[
            pltpu.VMEM((_PAIRS_PER_W, _IDX_PER_GATHER), jnp.int32),
            pltpu.VMEM((_IDX_PER_GATHER, EMBED_DIM), jnp.float32),
            pltpu.VMEM((_IDX_PER_GATHER, EMBED_DIM), jnp.float32),
            pltpu.VMEM((_ROWS_PER_W, EMBED_DIM), jnp.float32),
            pltpu.SemaphoreType.DMA,
            pltpu.SemaphoreType.DMA,
        ],
    )


_BM = 512


def _tc_head_body(x_ref, wt_ref, b_ref, o_ref):
    logits = (
        jnp.dot(x_ref[...], wt_ref[...], preferred_element_type=jnp.float32)
        + b_ref[...]
    )
    m = jnp.max(logits, axis=-1, keepdims=True)
    e = jnp.exp(logits - m)
    o_ref[...] = e / jnp.sum(e, axis=-1, keepdims=True)


@functools.partial(jax.jit, static_argnames=())
def _run(x, emb_table, W, b):
    x_pairs = jnp.reshape(x.astype(jnp.int32), (BATCH // 2, _IDX_PER_GATHER))
    pooled = _make_sc_pool()(x_pairs, emb_table)

    wt = jnp.zeros((EMBED_DIM, _PAD_CLASSES), jnp.float32)
    wt = wt.at[:, :NUM_CLASSES].set(W.T)
    bp = jnp.full((1, _PAD_CLASSES), -1e30, jnp.float32)
    bp = bp.at[0, :NUM_CLASSES].set(b)

    out = pl.pallas_call(
        _tc_head_body,
        grid=(BATCH // _BM,),
        in_specs=[
            pl.BlockSpec((_BM, EMBED_DIM), lambda i: (i, 0)),
            pl.BlockSpec((EMBED_DIM, _PAD_CLASSES), lambda i: (0, 0)),
            pl.BlockSpec((1, _PAD_CLASSES), lambda i: (0, 0)),
        ],
        out_specs=pl.BlockSpec((_BM, _PAD_CLASSES), lambda i: (i, 0)),
        out_shape=jax.ShapeDtypeStruct((BATCH, _PAD_CLASSES), jnp.float32),
    )(pooled, wt, bp)
    return out[:, :NUM_CLASSES]


def kernel(x, emb_table, W, b):
    return _run(x, emb_table, W, b)
